# baseline (speedup 1.0000x reference)
import functools

import jax
import jax.numpy as jnp
from jax import lax
from jax.experimental import pallas as pl
from jax.experimental.pallas import tpu as pltpu
from jax.experimental.pallas import tpu_sc as plsc

T = 2048
D = 1024
E = 16
K = 2
F = 1024
FS = 1024
C = 384
RSF = 1.0

BT = 256
NBLK = T // BT
NSLOT = E * C + 8
DUMP = E * C
NW = 32
TPW = T // NW
DP = D // 2


def _pack_bf16(xb):
    n2 = xb.shape[1] // 2
    h = lax.bitcast_convert_type(xb[:, :n2], jnp.uint16).astype(jnp.uint32)
    lo = lax.bitcast_convert_type(xb[:, n2:], jnp.uint16).astype(jnp.uint32)
    return lax.bitcast_convert_type((h << 16) | lo, jnp.float32)


def _unpack_bf16(p):
    u = lax.bitcast_convert_type(p, jnp.uint32)
    h = lax.bitcast_convert_type((u >> 16).astype(jnp.uint16), jnp.bfloat16)
    lo = lax.bitcast_convert_type((u & 0xFFFF).astype(jnp.uint16), jnp.bfloat16)
    return jnp.concatenate([h, lo], axis=1)


def _router_body(x_ref, gw_ref, bias_ref,
                 d0_ref, d1_ref, w0_ref, w1_ref, xp_ref, base_ref):
    pid = pl.program_id(0)

    @pl.when(pid == 0)
    def _():
        base_ref[...] = jnp.zeros_like(base_ref)

    x = x_ref[...]
    xp_ref[...] = _pack_bf16(x.astype(jnp.bfloat16))
    gw = gw_ref[...]
    logits = lax.dot_general(x, gw, (((1,), (1,)), ((), ())),
                             preferred_element_type=jnp.float32)
    scores = jax.nn.sigmoid(logits)
    choice = scores + bias_ref[...]

    ie = lax.broadcasted_iota(jnp.int32, (BT, E), 1)
    neg = jnp.float32(-jnp.inf)

    m1 = jnp.max(choice, axis=1, keepdims=True)
    i1 = jnp.min(jnp.where(choice == m1, ie, E), axis=1, keepdims=True)
    oh1 = ie == i1
    choice2 = jnp.where(oh1, neg, choice)
    m2 = jnp.max(choice2, axis=1, keepdims=True)
    i2 = jnp.min(jnp.where(choice2 == m2, ie, E), axis=1, keepdims=True)
    oh2 = ie == i2

    s1 = jnp.sum(jnp.where(oh1, scores, 0.0), axis=1, keepdims=True)
    s2 = jnp.sum(jnp.where(oh2, scores, 0.0), axis=1, keepdims=True)
    denom = s1 + s2 + 1e-20
    w1 = s1 / denom * RSF
    w2 = s2 / denom * RSF

    oh = oh1.astype(jnp.float32) + oh2.astype(jnp.float32)
    ir = lax.broadcasted_iota(jnp.int32, (BT, BT), 0)
    ic = lax.broadcasted_iota(jnp.int32, (BT, BT), 1)
    tri = (ic < ir).astype(jnp.float32)
    prefix = lax.dot_general(tri, oh, (((1,), (0,)), ((), ())),
                             preferred_element_type=jnp.float32)
    base = base_ref[...]
    posmat = base + prefix
    base_ref[...] = base + jnp.sum(oh, axis=0, keepdims=True)

    p1 = jnp.sum(jnp.where(oh1, posmat, 0.0), axis=1, keepdims=True).astype(jnp.int32)
    p2 = jnp.sum(jnp.where(oh2, posmat, 0.0), axis=1, keepdims=True).astype(jnp.int32)

    keep1 = p1 < C
    keep2 = p2 < C
    d0_ref[...] = jnp.where(keep1, i1 * C + p1, DUMP)
    d1_ref[...] = jnp.where(keep2, i2 * C + p2, DUMP)
    w0_ref[...] = jnp.where(keep1, w1, 0.0)
    w1_ref[...] = jnp.where(keep2, w2, 0.0)


def _router(x, gw, bias2d):
    call = pl.pallas_call(
        _router_body,
        grid=(NBLK,),
        in_specs=[
            pl.BlockSpec((BT, D), lambda i: (i, 0)),
            pl.BlockSpec((E, D), lambda i: (0, 0)),
            pl.BlockSpec((1, E), lambda i: (0, 0)),
        ],
        out_specs=[pl.BlockSpec((BT, 1), lambda i: (i, 0))] * 4
        + [pl.BlockSpec((BT, DP), lambda i: (i, 0))],
        out_shape=[jax.ShapeDtypeStruct((T, 1), jnp.int32)] * 2
        + [jax.ShapeDtypeStruct((T, 1), jnp.float32)] * 2
        + [jax.ShapeDtypeStruct((T, DP), jnp.float32)],
        scratch_shapes=[pltpu.VMEM((1, E), jnp.float32)],
    )
    return call(x, gw, bias2d)


def _sc_dispatch(xp, d0, d1, w0, w1):
    mesh = plsc.VectorSubcoreMesh(core_axis_name="c", subcore_axis_name="s")

    @functools.partial(
        pl.kernel,
        out_type=[
            jax.ShapeDtypeStruct((NSLOT, DP), jnp.float32),
            jax.ShapeDtypeStruct((NSLOT,), jnp.int32),
            jax.ShapeDtypeStruct((NSLOT,), jnp.float32),
        ],
        mesh=mesh,
        compiler_params=pltpu.CompilerParams(needs_layout_passes=False),
        scratch_types=[
            pltpu.VMEM((TPW,), jnp.int32),
            pltpu.VMEM((TPW, DP), jnp.float32),
            pltpu.VMEM((T,), jnp.int32),
            pltpu.VMEM((T,), jnp.int32),
            pltpu.VMEM((T,), jnp.float32),
            pltpu.VMEM((T,), jnp.float32),
            pltpu.VMEM((NSLOT,), jnp.int32),
            pltpu.VMEM((NSLOT,), jnp.float32),
            pltpu.SemaphoreType.DMA,
        ],
    )
    def k(xp_hbm, d0_hbm, d1_hbm, w0_hbm, w1_hbm,
          disp_hbm, invt_hbm, invw_hbm,
          idx_v, rows_v, d0_v, d1_v, w0_v, w1_v, invt_v, invw_v, sem):
        wid = lax.axis_index("s") * 2 + lax.axis_index("c")
        base = wid * TPW
        pltpu.sync_copy(xp_hbm.at[pl.ds(base, TPW)], rows_v)
        pltpu.sync_copy(d0_hbm.at[pl.ds(base, TPW)], idx_v)
        pltpu.async_copy(rows_v, disp_hbm.at[idx_v], sem).wait()
        pltpu.sync_copy(d1_hbm.at[pl.ds(base, TPW)], idx_v)
        pltpu.async_copy(rows_v, disp_hbm.at[idx_v], sem).wait()

        @pl.when(wid == 0)
        def _():
            pltpu.sync_copy(d0_hbm, d0_v)
            pltpu.sync_copy(d1_hbm, d1_v)
            pltpu.sync_copy(w0_hbm, w0_v)
            pltpu.sync_copy(w1_hbm, w1_v)

            def zbody(i, c):
                invw_v[pl.ds(i * 16, 16)] = jnp.zeros((16,), jnp.float32)
                invt_v[pl.ds(i * 16, 16)] = jnp.zeros((16,), jnp.int32)
                return c

            lax.fori_loop(0, NSLOT // 16, zbody, 0)

            def sbody(i, c):
                sl = pl.ds(i * 16, 16)
                toks = lax.iota(jnp.int32, 16) + i * 16
                idx0 = d0_v[sl]
                plsc.store_scatter(invw_v, [idx0], w0_v[sl])
                plsc.store_scatter(invt_v, [idx0], toks)
                idx1 = d1_v[sl]
                plsc.store_scatter(invw_v, [idx1], w1_v[sl])
                plsc.store_scatter(invt_v, [idx1], toks)
                return c

            lax.fori_loop(0, T // 16, sbody, 0)
            pltpu.sync_copy(invt_v, invt_hbm)
            pltpu.sync_copy(invw_v, invw_hbm)

    return k(xp, d0, d1, w0, w1)


def _sharedffn_body(x_ref, wgu_ref, wdn_ref, o_ref):
    xb = x_ref[...].astype(jnp.bfloat16)
    h = jnp.dot(xb, wgu_ref[...].astype(jnp.bfloat16),
                preferred_element_type=jnp.float32)
    g = h[:, :FS]
    u = h[:, FS:]
    act = (g * jax.nn.sigmoid(g) * u).astype(jnp.bfloat16)
    o_ref[...] = jnp.dot(act, wdn_ref[...].astype(jnp.bfloat16),
                         preferred_element_type=jnp.float32)


def _sharedffn(x, swgu, swdn):
    call = pl.pallas_call(
        _sharedffn_body,
        grid=(NBLK,),
        in_specs=[
            pl.BlockSpec((BT, D), lambda i: (i, 0)),
            pl.BlockSpec((D, 2 * FS), lambda i: (0, 0)),
            pl.BlockSpec((FS, D), lambda i: (0, 0)),
        ],
        out_specs=pl.BlockSpec((BT, D), lambda i: (i, 0)),
        out_shape=jax.ShapeDtypeStruct((T, D), jnp.float32),
    )
    return call(x, swgu, swdn)


def _ffn_body(disp_ref, wgu_ref, wdn_ref, invt_ref, invw_ref, sh_ref, out_ref):
    e = pl.program_id(0)
    xb = _unpack_bf16(disp_ref[...])
    wgu = wgu_ref[0].astype(jnp.bfloat16)
    h = jnp.dot(xb, wgu, preferred_element_type=jnp.float32)
    g = h[:, :F]
    u = h[:, F:]
    act = (g * jax.nn.sigmoid(g) * u).astype(jnp.bfloat16)
    wdn = wdn_ref[0].astype(jnp.bfloat16)
    eo = jnp.dot(act, wdn, preferred_element_type=jnp.float32)
    eo = jnp.where(jnp.abs(eo) < jnp.float32(3e38), eo, 0.0)
    eo_b = eo.astype(jnp.bfloat16)

    it = lax.broadcasted_iota(jnp.int32, (C, T), 1)
    oh = jnp.where(it == invt_ref[...], invw_ref[...], 0.0).astype(jnp.bfloat16)
    contrib = lax.dot_general(oh, eo_b, (((0,), (0,)), ((), ())),
                              preferred_element_type=jnp.float32)

    @pl.when(e == 0)
    def _():
        out_ref[...] = contrib

    @pl.when(e > 0)
    def _():
        out_ref[...] += contrib

    @pl.when(e == E - 1)
    def _():
        out_ref[...] += sh_ref[...]


def _ffn_combine(disp, w_gate_up, w_down, invt, invw, sh):
    call = pl.pallas_call(
        _ffn_body,
        grid=(E,),
        in_specs=[
            pl.BlockSpec((C, DP), lambda e: (e, 0)),
            pl.BlockSpec((1, D, 2 * F), lambda e: (e, 0, 0)),
            pl.BlockSpec((1, F, D), lambda e: (e, 0, 0)),
            pl.BlockSpec((C, 1), lambda e: (e, 0)),
            pl.BlockSpec((C, 1), lambda e: (e, 0)),
            pl.BlockSpec((T, D), lambda e: (0, 0)),
        ],
        out_specs=pl.BlockSpec((T, D), lambda e: (0, 0)),
        out_shape=jax.ShapeDtypeStruct((T, D), jnp.float32),
    )
    return call(disp, w_gate_up, w_down, invt, invw, sh)


def kernel(hidden_states, gate_weight, e_score_correction_bias,
           w_gate_up, w_down, shared_w_gate_up, shared_w_down):
    x = hidden_states
    bias2d = e_score_correction_bias.reshape(1, E)
    d0, d1, w0, w1, xp = _router(x, gate_weight, bias2d)
    disp, invt, invw = _sc_dispatch(xp, d0.reshape(T), d1.reshape(T),
                                    w0.reshape(T), w1.reshape(T))
    sh = _sharedffn(x, shared_w_gate_up, shared_w_down)
    return _ffn_combine(disp, w_gate_up, w_down,
                        invt.reshape(NSLOT, 1), invw.reshape(NSLOT, 1), sh)


# speedup vs baseline: 1.2862x; 1.2862x over previous
import functools

import jax
import jax.numpy as jnp
from jax import lax
from jax.experimental import pallas as pl
from jax.experimental.pallas import tpu as pltpu
from jax.experimental.pallas import tpu_sc as plsc

T = 2048
D = 1024
E = 16
K = 2
F = 1024
FS = 1024
C = 384
RSF = 1.0

BT = 256
NBLK = T // BT
NSLOT = E * C + 8
DUMP = E * C
NW = 32
TPW = T // NW
DP = D // 2
TS = T // E
BTF = 512


def _pack_bf16(xb):
    n2 = xb.shape[1] // 2
    h = lax.bitcast_convert_type(xb[:, :n2], jnp.uint16).astype(jnp.uint32)
    lo = lax.bitcast_convert_type(xb[:, n2:], jnp.uint16).astype(jnp.uint32)
    return lax.bitcast_convert_type((h << 16) | lo, jnp.float32)


def _unpack_bf16(p):
    u = lax.bitcast_convert_type(p, jnp.uint32)
    h = lax.bitcast_convert_type((u >> 16).astype(jnp.uint16), jnp.bfloat16)
    lo = lax.bitcast_convert_type((u & 0xFFFF).astype(jnp.uint16), jnp.bfloat16)
    return jnp.concatenate([h, lo], axis=1)


def _router_body(x_ref, gw_ref, bias_ref,
                 idx4_ref, w0_ref, w1_ref, xp_ref, base_ref):
    pid = pl.program_id(0)

    @pl.when(pid == 0)
    def _():
        base_ref[...] = jnp.zeros_like(base_ref)

    x = x_ref[...]
    xp_ref[...] = _pack_bf16(x.astype(jnp.bfloat16))
    gw = gw_ref[...]
    logits = lax.dot_general(x, gw, (((1,), (1,)), ((), ())),
                             preferred_element_type=jnp.float32)
    scores = jax.nn.sigmoid(logits)
    choice = scores + bias_ref[...]

    ie = lax.broadcasted_iota(jnp.int32, (BT, E), 1)
    neg = jnp.float32(-jnp.inf)

    m1 = jnp.max(choice, axis=1, keepdims=True)
    i1 = jnp.min(jnp.where(choice == m1, ie, E), axis=1, keepdims=True)
    oh1 = ie == i1
    choice2 = jnp.where(oh1, neg, choice)
    m2 = jnp.max(choice2, axis=1, keepdims=True)
    i2 = jnp.min(jnp.where(choice2 == m2, ie, E), axis=1, keepdims=True)
    oh2 = ie == i2

    s1 = jnp.sum(jnp.where(oh1, scores, 0.0), axis=1, keepdims=True)
    s2 = jnp.sum(jnp.where(oh2, scores, 0.0), axis=1, keepdims=True)
    denom = s1 + s2 + 1e-20
    w1 = s1 / denom * RSF
    w2 = s2 / denom * RSF

    oh = oh1.astype(jnp.float32) + oh2.astype(jnp.float32)
    ir = lax.broadcasted_iota(jnp.int32, (BT, BT), 0)
    ic = lax.broadcasted_iota(jnp.int32, (BT, BT), 1)
    tri = (ic < ir).astype(jnp.float32)
    prefix = lax.dot_general(tri, oh, (((1,), (0,)), ((), ())),
                             preferred_element_type=jnp.float32)
    base = base_ref[...]
    posmat = base + prefix
    base_ref[...] = base + jnp.sum(oh, axis=0, keepdims=True)

    p1 = jnp.sum(jnp.where(oh1, posmat, 0.0), axis=1, keepdims=True).astype(jnp.int32)
    p2 = jnp.sum(jnp.where(oh2, posmat, 0.0), axis=1, keepdims=True).astype(jnp.int32)

    keep1 = p1 < C
    keep2 = p2 < C
    d0 = jnp.where(keep1, i1 * C + p1, DUMP)
    d1 = jnp.where(keep2, i2 * C + p2, DUMP)
    r0 = i1 * C + jnp.minimum(p1, C - 1)
    r1 = i2 * C + jnp.minimum(p2, C - 1)
    idx4_ref[...] = jnp.concatenate([d0, d1, r0, r1], axis=1)
    w0_ref[...] = jnp.where(keep1, w1, 0.0)
    w1_ref[...] = jnp.where(keep2, w2, 0.0)


def _router(x, gw, bias2d):
    call = pl.pallas_call(
        _router_body,
        grid=(NBLK,),
        in_specs=[
            pl.BlockSpec((BT, D), lambda i: (i, 0)),
            pl.BlockSpec((E, D), lambda i: (0, 0)),
            pl.BlockSpec((1, E), lambda i: (0, 0)),
        ],
        out_specs=[pl.BlockSpec((BT, 4), lambda i: (i, 0)),
                   pl.BlockSpec((BT, 1), lambda i: (i, 0)),
                   pl.BlockSpec((BT, 1), lambda i: (i, 0)),
                   pl.BlockSpec((BT, DP), lambda i: (i, 0))],
        out_shape=[jax.ShapeDtypeStruct((T, 4), jnp.int32),
                   jax.ShapeDtypeStruct((T, 1), jnp.float32),
                   jax.ShapeDtypeStruct((T, 1), jnp.float32),
                   jax.ShapeDtypeStruct((T, DP), jnp.float32)],
        scratch_shapes=[pltpu.VMEM((1, E), jnp.float32)],
    )
    return call(x, gw, bias2d)


def _deinterleave_col(d4_v, col, out_ref):
    cols = jnp.full((16,), col, jnp.int32)
    for g in range(TPW // 16):
        rows = lax.iota(jnp.int32, 16) + 16 * g
        out_ref[pl.ds(16 * g, 16)] = plsc.load_gather(d4_v, [rows, cols])


def _sc_dispatch(xp, idx4):
    mesh = plsc.VectorSubcoreMesh(core_axis_name="c", subcore_axis_name="s")

    @functools.partial(
        pl.kernel,
        out_type=jax.ShapeDtypeStruct((NSLOT, DP), jnp.float32),
        mesh=mesh,
        compiler_params=pltpu.CompilerParams(needs_layout_passes=False),
        scratch_types=[
            pltpu.VMEM((TPW, 4), jnp.int32),
            pltpu.VMEM((TPW,), jnp.int32),
            pltpu.VMEM((TPW,), jnp.int32),
            pltpu.VMEM((TPW, DP), jnp.float32),
            pltpu.SemaphoreType.DMA,
            pltpu.SemaphoreType.DMA,
        ],
    )
    def k(xp_hbm, idx4_hbm, disp_hbm, d4_v, idx0_v, idx1_v, rows_v, sem0, sem1):
        wid = lax.axis_index("s") * 2 + lax.axis_index("c")
        base = wid * TPW
        pltpu.sync_copy(idx4_hbm.at[pl.ds(base, TPW)], d4_v)
        pltpu.sync_copy(xp_hbm.at[pl.ds(base, TPW)], rows_v)
        _deinterleave_col(d4_v, 0, idx0_v)
        _deinterleave_col(d4_v, 1, idx1_v)
        c0 = pltpu.async_copy(rows_v, disp_hbm.at[idx0_v], sem0)
        c1 = pltpu.async_copy(rows_v, disp_hbm.at[idx1_v], sem1)
        c0.wait()
        c1.wait()

    return k(xp, idx4)


def _ffn_body(disp_ref, wgu_ref, wdn_ref, xs_ref, swgu_ref, swdn_ref,
              eo_ref, sh_ref, swgu_bf, swdn_bf):
    e = pl.program_id(0)

    @pl.when(e == 0)
    def _():
        swgu_bf[...] = swgu_ref[...].astype(jnp.bfloat16)
        swdn_bf[...] = swdn_ref[...].astype(jnp.bfloat16)

    xb = _unpack_bf16(disp_ref[...])
    wgu = wgu_ref[0].astype(jnp.bfloat16)
    h = jnp.dot(xb, wgu, preferred_element_type=jnp.float32)
    g = h[:, :F]
    u = h[:, F:]
    act = (g * jax.nn.sigmoid(g) * u).astype(jnp.bfloat16)
    wdn = wdn_ref[0].astype(jnp.bfloat16)
    eo = jnp.dot(act, wdn, preferred_element_type=jnp.float32)
    eo_ref[...] = _pack_bf16(eo.astype(jnp.bfloat16))

    xs = xs_ref[...].astype(jnp.bfloat16)
    hs = jnp.dot(xs, swgu_bf[...], preferred_element_type=jnp.float32)
    gs = hs[:, :FS]
    us = hs[:, FS:]
    acts = (gs * jax.nn.sigmoid(gs) * us).astype(jnp.bfloat16)
    sh_ref[...] = jnp.dot(acts, swdn_bf[...],
                          preferred_element_type=jnp.float32)


def _ffn_shared(disp, w_gate_up, w_down, x, swgu, swdn):
    call = pl.pallas_call(
        _ffn_body,
        grid=(E,),
        in_specs=[
            pl.BlockSpec((C, DP), lambda e: (e, 0)),
            pl.BlockSpec((1, D, 2 * F), lambda e: (e, 0, 0)),
            pl.BlockSpec((1, F, D), lambda e: (e, 0, 0)),
            pl.BlockSpec((TS, D), lambda e: (e, 0)),
            pl.BlockSpec((D, 2 * FS), lambda e: (0, 0)),
            pl.BlockSpec((FS, D), lambda e: (0, 0)),
        ],
        out_specs=[pl.BlockSpec((C, DP), lambda e: (e, 0)),
                   pl.BlockSpec((TS, D), lambda e: (e, 0))],
        out_shape=[jax.ShapeDtypeStruct((E * C, DP), jnp.float32),
                   jax.ShapeDtypeStruct((T, D), jnp.float32)],
        scratch_shapes=[pltpu.VMEM((D, 2 * FS), jnp.bfloat16),
                        pltpu.VMEM((FS, D), jnp.bfloat16)],
    )
    return call(disp, w_gate_up, w_down, x, swgu, swdn)


def _sc_combine(eo, idx4):
    mesh = plsc.VectorSubcoreMesh(core_axis_name="c", subcore_axis_name="s")

    @functools.partial(
        pl.kernel,
        out_type=[jax.ShapeDtypeStruct((T, DP), jnp.float32)] * 2,
        mesh=mesh,
        compiler_params=pltpu.CompilerParams(needs_layout_passes=False),
        scratch_types=[
            pltpu.VMEM((TPW, 4), jnp.int32),
            pltpu.VMEM((TPW,), jnp.int32),
            pltpu.VMEM((TPW,), jnp.int32),
            pltpu.VMEM((TPW, DP), jnp.float32),
            pltpu.VMEM((TPW, DP), jnp.float32),
            pltpu.SemaphoreType.DMA,
            pltpu.SemaphoreType.DMA,
        ],
    )
    def k(eo_hbm, idx4_hbm, g0_hbm, g1_hbm,
          d4_v, idx0_v, idx1_v, rows0_v, rows1_v, sem0, sem1):
        wid = lax.axis_index("s") * 2 + lax.axis_index("c")
        base = wid * TPW
        pltpu.sync_copy(idx4_hbm.at[pl.ds(base, TPW)], d4_v)
        _deinterleave_col(d4_v, 2, idx0_v)
        _deinterleave_col(d4_v, 3, idx1_v)
        c0 = pltpu.async_copy(eo_hbm.at[idx0_v], rows0_v, sem0)
        c1 = pltpu.async_copy(eo_hbm.at[idx1_v], rows1_v, sem1)
        c0.wait()
        pltpu.sync_copy(rows0_v, g0_hbm.at[pl.ds(base, TPW)])
        c1.wait()
        pltpu.sync_copy(rows1_v, g1_hbm.at[pl.ds(base, TPW)])

    return k(eo, idx4)


def _fin_body(sh_ref, g0_ref, g1_ref, w0_ref, w1_ref, o_ref):
    o_ref[...] = (sh_ref[...]
                  + w0_ref[...] * _unpack_bf16(g0_ref[...]).astype(jnp.float32)
                  + w1_ref[...] * _unpack_bf16(g1_ref[...]).astype(jnp.float32))


def _final_combine(sh, g0, g1, w0, w1):
    call = pl.pallas_call(
        _fin_body,
        grid=(T // BTF,),
        in_specs=[
            pl.BlockSpec((BTF, D), lambda i: (i, 0)),
            pl.BlockSpec((BTF, DP), lambda i: (i, 0)),
            pl.BlockSpec((BTF, DP), lambda i: (i, 0)),
            pl.BlockSpec((BTF, 1), lambda i: (i, 0)),
            pl.BlockSpec((BTF, 1), lambda i: (i, 0)),
        ],
        out_specs=pl.BlockSpec((BTF, D), lambda i: (i, 0)),
        out_shape=jax.ShapeDtypeStruct((T, D), jnp.float32),
    )
    return call(sh, g0, g1, w0, w1)


def kernel(hidden_states, gate_weight, e_score_correction_bias,
           w_gate_up, w_down, shared_w_gate_up, shared_w_down):
    x = hidden_states
    bias2d = e_score_correction_bias.reshape(1, E)
    idx4, w0, w1, xp = _router(x, gate_weight, bias2d)
    disp = _sc_dispatch(xp, idx4)
    eo, sh = _ffn_shared(disp, w_gate_up, w_down, x,
                         shared_w_gate_up, shared_w_down)
    g0, g1 = _sc_combine(eo, idx4)
    return _final_combine(sh, g0, g1, w0, w1)


# speedup vs baseline: 1.3118x; 1.0200x over previous
import functools

import jax
import jax.numpy as jnp
from jax import lax
from jax.experimental import pallas as pl
from jax.experimental.pallas import tpu as pltpu
from jax.experimental.pallas import tpu_sc as plsc

T = 2048
D = 1024
E = 16
K = 2
F = 1024
FS = 1024
C = 384
RSF = 1.0

BT = 512
NBLK = T // BT
NSLOT = E * C + 8
DUMP = E * C
NW = 32
TPW = T // NW
DP = D // 2
TS = T // E
BTF = 512


def _pack_bf16(xb):
    n2 = xb.shape[1] // 2
    h = lax.bitcast_convert_type(xb[:, :n2], jnp.uint16).astype(jnp.uint32)
    lo = lax.bitcast_convert_type(xb[:, n2:], jnp.uint16).astype(jnp.uint32)
    return lax.bitcast_convert_type((h << 16) | lo, jnp.float32)


def _unpack_bf16(p):
    u = lax.bitcast_convert_type(p, jnp.uint32)
    h = lax.bitcast_convert_type((u >> 16).astype(jnp.uint16), jnp.bfloat16)
    lo = lax.bitcast_convert_type((u & 0xFFFF).astype(jnp.uint16), jnp.bfloat16)
    return jnp.concatenate([h, lo], axis=1)


def _router_body(x_ref, gw_ref, bias_ref,
                 idx4_ref, w0_ref, w1_ref, xp_ref, base_ref):
    pid = pl.program_id(0)

    @pl.when(pid == 0)
    def _():
        base_ref[...] = jnp.zeros_like(base_ref)

    x = x_ref[...]
    xp_ref[...] = _pack_bf16(x.astype(jnp.bfloat16))
    gw = gw_ref[...]
    logits = lax.dot_general(x, gw, (((1,), (1,)), ((), ())),
                             preferred_element_type=jnp.float32)
    scores = jax.nn.sigmoid(logits)
    choice = scores + bias_ref[...]

    ie = lax.broadcasted_iota(jnp.int32, (BT, E), 1)
    neg = jnp.float32(-jnp.inf)

    m1 = jnp.max(choice, axis=1, keepdims=True)
    i1 = jnp.min(jnp.where(choice == m1, ie, E), axis=1, keepdims=True)
    oh1 = ie == i1
    choice2 = jnp.where(oh1, neg, choice)
    m2 = jnp.max(choice2, axis=1, keepdims=True)
    i2 = jnp.min(jnp.where(choice2 == m2, ie, E), axis=1, keepdims=True)
    oh2 = ie == i2

    s1 = jnp.sum(jnp.where(oh1, scores, 0.0), axis=1, keepdims=True)
    s2 = jnp.sum(jnp.where(oh2, scores, 0.0), axis=1, keepdims=True)
    denom = s1 + s2 + 1e-20
    w1 = s1 / denom * RSF
    w2 = s2 / denom * RSF

    oh = oh1.astype(jnp.float32) + oh2.astype(jnp.float32)
    ir = lax.broadcasted_iota(jnp.int32, (BT, BT), 0)
    ic = lax.broadcasted_iota(jnp.int32, (BT, BT), 1)
    tri = (ic < ir).astype(jnp.float32)
    prefix = lax.dot_general(tri, oh, (((1,), (0,)), ((), ())),
                             preferred_element_type=jnp.float32)
    base = base_ref[...]
    posmat = base + prefix
    base_ref[...] = base + jnp.sum(oh, axis=0, keepdims=True)

    p1 = jnp.sum(jnp.where(oh1, posmat, 0.0), axis=1, keepdims=True).astype(jnp.int32)
    p2 = jnp.sum(jnp.where(oh2, posmat, 0.0), axis=1, keepdims=True).astype(jnp.int32)

    keep1 = p1 < C
    keep2 = p2 < C
    d0 = jnp.where(keep1, i1 * C + p1, DUMP)
    d1 = jnp.where(keep2, i2 * C + p2, DUMP)
    r0 = i1 * C + jnp.minimum(p1, C - 1)
    r1 = i2 * C + jnp.minimum(p2, C - 1)
    idx4_ref[...] = jnp.concatenate([d0, d1, r0, r1], axis=1)
    w0_ref[...] = jnp.where(keep1, w1, 0.0)
    w1_ref[...] = jnp.where(keep2, w2, 0.0)


def _router(x, gw, bias2d):
    call = pl.pallas_call(
        _router_body,
        grid=(NBLK,),
        in_specs=[
            pl.BlockSpec((BT, D), lambda i: (i, 0)),
            pl.BlockSpec((E, D), lambda i: (0, 0)),
            pl.BlockSpec((1, E), lambda i: (0, 0)),
        ],
        out_specs=[pl.BlockSpec((BT, 4), lambda i: (i, 0)),
                   pl.BlockSpec((BT, 1), lambda i: (i, 0)),
                   pl.BlockSpec((BT, 1), lambda i: (i, 0)),
                   pl.BlockSpec((BT, DP), lambda i: (i, 0))],
        out_shape=[jax.ShapeDtypeStruct((T, 4), jnp.int32),
                   jax.ShapeDtypeStruct((T, 1), jnp.float32),
                   jax.ShapeDtypeStruct((T, 1), jnp.float32),
                   jax.ShapeDtypeStruct((T, DP), jnp.float32)],
        scratch_shapes=[pltpu.VMEM((1, E), jnp.float32)],
    )
    return call(x, gw, bias2d)


def _deinterleave_col(d4_v, col, out_ref):
    cols = jnp.full((16,), col, jnp.int32)
    for g in range(TPW // 16):
        rows = lax.iota(jnp.int32, 16) + 16 * g
        out_ref[pl.ds(16 * g, 16)] = plsc.load_gather(d4_v, [rows, cols])


def _sc_dispatch(xp, idx4):
    mesh = plsc.VectorSubcoreMesh(core_axis_name="c", subcore_axis_name="s")

    @functools.partial(
        pl.kernel,
        out_type=jax.ShapeDtypeStruct((NSLOT, DP), jnp.float32),
        mesh=mesh,
        compiler_params=pltpu.CompilerParams(needs_layout_passes=False),
        scratch_types=[
            pltpu.VMEM((TPW, 4), jnp.int32),
            pltpu.VMEM((TPW,), jnp.int32),
            pltpu.VMEM((TPW,), jnp.int32),
            pltpu.VMEM((TPW, DP), jnp.float32),
            pltpu.SemaphoreType.DMA,
            pltpu.SemaphoreType.DMA,
        ],
    )
    def k(xp_hbm, idx4_hbm, disp_hbm, d4_v, idx0_v, idx1_v, rows_v, sem0, sem1):
        wid = lax.axis_index("s") * 2 + lax.axis_index("c")
        base = wid * TPW
        pltpu.sync_copy(idx4_hbm.at[pl.ds(base, TPW)], d4_v)
        pltpu.sync_copy(xp_hbm.at[pl.ds(base, TPW)], rows_v)
        _deinterleave_col(d4_v, 0, idx0_v)
        _deinterleave_col(d4_v, 1, idx1_v)
        c0 = pltpu.async_copy(rows_v, disp_hbm.at[idx0_v], sem0)
        c1 = pltpu.async_copy(rows_v, disp_hbm.at[idx1_v], sem1)
        c0.wait()
        c1.wait()

    return k(xp, idx4)


def _ffn_body(disp_ref, wgu_ref, wdn_ref, xs_ref, swgu_ref, swdn_ref,
              eo_ref, sh_ref, swgu_bf, swdn_bf):
    e = pl.program_id(0)

    @pl.when(e == 0)
    def _():
        swgu_bf[...] = swgu_ref[...].astype(jnp.bfloat16)
        swdn_bf[...] = swdn_ref[...].astype(jnp.bfloat16)

    xb = _unpack_bf16(disp_ref[...])
    wgu = wgu_ref[0].astype(jnp.bfloat16)
    h = jnp.dot(xb, wgu, preferred_element_type=jnp.float32)
    g = h[:, :F]
    u = h[:, F:]
    act = (g * jax.nn.sigmoid(g) * u).astype(jnp.bfloat16)
    wdn = wdn_ref[0].astype(jnp.bfloat16)
    eo = jnp.dot(act, wdn, preferred_element_type=jnp.float32)
    eo_ref[...] = _pack_bf16(eo.astype(jnp.bfloat16))

    xs = xs_ref[...].astype(jnp.bfloat16)
    hs = jnp.dot(xs, swgu_bf[...], preferred_element_type=jnp.float32)
    gs = hs[:, :FS]
    us = hs[:, FS:]
    acts = (gs * jax.nn.sigmoid(gs) * us).astype(jnp.bfloat16)
    sh_ref[...] = jnp.dot(acts, swdn_bf[...],
                          preferred_element_type=jnp.float32)


def _ffn_shared(disp, w_gate_up, w_down, x, swgu, swdn):
    call = pl.pallas_call(
        _ffn_body,
        grid=(E,),
        in_specs=[
            pl.BlockSpec((C, DP), lambda e: (e, 0)),
            pl.BlockSpec((1, D, 2 * F), lambda e: (e, 0, 0)),
            pl.BlockSpec((1, F, D), lambda e: (e, 0, 0)),
            pl.BlockSpec((TS, D), lambda e: (e, 0)),
            pl.BlockSpec((D, 2 * FS), lambda e: (0, 0)),
            pl.BlockSpec((FS, D), lambda e: (0, 0)),
        ],
        out_specs=[pl.BlockSpec((C, DP), lambda e: (e, 0)),
                   pl.BlockSpec((TS, D), lambda e: (e, 0))],
        out_shape=[jax.ShapeDtypeStruct((E * C, DP), jnp.float32),
                   jax.ShapeDtypeStruct((T, D), jnp.float32)],
        scratch_shapes=[pltpu.VMEM((D, 2 * FS), jnp.bfloat16),
                        pltpu.VMEM((FS, D), jnp.bfloat16)],
    )
    return call(disp, w_gate_up, w_down, x, swgu, swdn)


def _sc_combine(eo, idx4):
    mesh = plsc.VectorSubcoreMesh(core_axis_name="c", subcore_axis_name="s")

    @functools.partial(
        pl.kernel,
        out_type=[jax.ShapeDtypeStruct((T, DP), jnp.float32)] * 2,
        mesh=mesh,
        compiler_params=pltpu.CompilerParams(needs_layout_passes=False),
        scratch_types=[
            pltpu.VMEM((TPW, 4), jnp.int32),
            pltpu.VMEM((TPW,), jnp.int32),
            pltpu.VMEM((TPW,), jnp.int32),
            pltpu.VMEM((TPW, DP), jnp.float32),
            pltpu.VMEM((TPW, DP), jnp.float32),
            pltpu.SemaphoreType.DMA,
            pltpu.SemaphoreType.DMA,
        ],
    )
    def k(eo_hbm, idx4_hbm, g0_hbm, g1_hbm,
          d4_v, idx0_v, idx1_v, rows0_v, rows1_v, sem0, sem1):
        wid = lax.axis_index("s") * 2 + lax.axis_index("c")
        base = wid * TPW
        pltpu.sync_copy(idx4_hbm.at[pl.ds(base, TPW)], d4_v)
        _deinterleave_col(d4_v, 2, idx0_v)
        _deinterleave_col(d4_v, 3, idx1_v)
        c0 = pltpu.async_copy(eo_hbm.at[idx0_v], rows0_v, sem0)
        c1 = pltpu.async_copy(eo_hbm.at[idx1_v], rows1_v, sem1)
        c0.wait()
        pltpu.sync_copy(rows0_v, g0_hbm.at[pl.ds(base, TPW)])
        c1.wait()
        pltpu.sync_copy(rows1_v, g1_hbm.at[pl.ds(base, TPW)])

    return k(eo, idx4)


def _fin_body(sh_ref, g0_ref, g1_ref, w0_ref, w1_ref, o_ref):
    o_ref[...] = (sh_ref[...]
                  + w0_ref[...] * _unpack_bf16(g0_ref[...]).astype(jnp.float32)
                  + w1_ref[...] * _unpack_bf16(g1_ref[...]).astype(jnp.float32))


def _final_combine(sh, g0, g1, w0, w1):
    call = pl.pallas_call(
        _fin_body,
        grid=(T // BTF,),
        in_specs=[
            pl.BlockSpec((BTF, D), lambda i: (i, 0)),
            pl.BlockSpec((BTF, DP), lambda i: (i, 0)),
            pl.BlockSpec((BTF, DP), lambda i: (i, 0)),
            pl.BlockSpec((BTF, 1), lambda i: (i, 0)),
            pl.BlockSpec((BTF, 1), lambda i: (i, 0)),
        ],
        out_specs=pl.BlockSpec((BTF, D), lambda i: (i, 0)),
        out_shape=jax.ShapeDtypeStruct((T, D), jnp.float32),
    )
    return call(sh, g0, g1, w0, w1)


def kernel(hidden_states, gate_weight, e_score_correction_bias,
           w_gate_up, w_down, shared_w_gate_up, shared_w_down):
    x = hidden_states
    bias2d = e_score_correction_bias.reshape(1, E)
    idx4, w0, w1, xp = _router(x, gate_weight, bias2d)
    disp = _sc_dispatch(xp, idx4)
    eo, sh = _ffn_shared(disp, w_gate_up, w_down, x,
                         shared_w_gate_up, shared_w_down)
    g0, g1 = _sc_combine(eo, idx4)
    return _final_combine(sh, g0, g1, w0, w1)
